# Initial kernel scaffold; baseline (speedup 1.0000x reference)
import jax
import jax.numpy as jnp
from jax.experimental import pallas as pl


def kernel(token_table, seg_table, g_table, b_W, b_b, h_W, h_b, sequence, segment_label, y):
    raise NotImplementedError("write your pallas kernel here")


# speedup vs baseline: 2.0262x; 2.0262x over previous
import functools

import jax
import jax.numpy as jnp
import numpy as np
from jax import lax
from jax.experimental import pallas as pl
from jax.experimental.pallas import tpu as pltpu
from jax.experimental.pallas import tpu_sc as plsc

VOCAB = 30522
D = 768
CLASS = 1000
CPAD = 1024
G = 128
B = 4
L = 2048
N = B * L

_NEG = -1e30


def _make_pe(seq_len, d_model):
    pos = np.arange(seq_len)[:, None].astype(np.float32)
    div = np.exp(np.arange(0, d_model, 2).astype(np.float32) * -(np.log(10000.0) / d_model))
    pe = np.zeros((seq_len, d_model), dtype=np.float32)
    pe[:, 0::2] = np.sin(pos * div)
    pe[:, 1::2] = np.cos(pos * div)
    return pe


def _sc_gather(token_table, g_table, seq_flat, y_flat):
    info = plsc.get_sparse_core_info()
    NC, NS = info.num_cores, info.num_subcores
    NW = NC * NS
    n_per_w = N // NW
    TOK_CH = 64
    Y_CH = 128

    mesh = plsc.VectorSubcoreMesh(core_axis_name="c", subcore_axis_name="s")

    @functools.partial(
        pl.kernel,
        mesh=mesh,
        out_type=(
            jax.ShapeDtypeStruct((N, D), jnp.float32),
            jax.ShapeDtypeStruct((N, G), jnp.float32),
        ),
        scratch_types=[
            pltpu.VMEM((n_per_w,), jnp.int32),
            pltpu.VMEM((n_per_w,), jnp.int32),
            pltpu.VMEM((TOK_CH, D), jnp.float32),
            pltpu.VMEM((Y_CH, G), jnp.float32),
            pltpu.SemaphoreType.DMA,
        ],
    )
    def k(tab_hbm, g_hbm, seq_hbm, y_hbm, tok_out, yemb_out,
          seq_v, y_v, tok_buf, y_buf, sem):
        wid = lax.axis_index("s") * NC + lax.axis_index("c")
        base = wid * n_per_w
        pltpu.sync_copy(seq_hbm.at[pl.ds(base, n_per_w)], seq_v)
        pltpu.sync_copy(y_hbm.at[pl.ds(base, n_per_w)], y_v)
        for c in range(n_per_w // TOK_CH):
            pltpu.async_copy(
                tab_hbm.at[seq_v.at[pl.ds(c * TOK_CH, TOK_CH)]], tok_buf, sem
            ).wait()
            pltpu.sync_copy(tok_buf, tok_out.at[pl.ds(base + c * TOK_CH, TOK_CH)])
        for c in range(n_per_w // Y_CH):
            pltpu.async_copy(
                g_hbm.at[y_v.at[pl.ds(c * Y_CH, Y_CH)]], y_buf, sem
            ).wait()
            pltpu.sync_copy(y_buf, yemb_out.at[pl.ds(base + c * Y_CH, Y_CH)])

    return k(token_table, g_table, seq_flat, y_flat)


_R = 256
_NBLK = N // _R


def _tc_body(tok_ref, pe_ref, s1h_ref, yemb_ref, y3_ref,
             bW_ref, bb_ref, hW_ref, hb_ref, segT_ref,
             x_ref, loss_ref):
    i = pl.program_id(0)
    seg = jnp.dot(s1h_ref[...], segT_ref[...], preferred_element_type=jnp.float32)
    x = tok_ref[...] + pe_ref[...] + seg
    x_ref[...] = x
    xb = jnp.dot(x, bW_ref[...], preferred_element_type=jnp.float32) + bb_ref[...]
    diff = xb - yemb_ref[...]
    ass_part = jnp.sum(diff * diff)
    logits = jnp.dot(yemb_ref[...], hW_ref[...], preferred_element_type=jnp.float32) + hb_ref[...]
    m = jnp.max(logits, axis=-1, keepdims=True)
    lse = jnp.log(jnp.sum(jnp.exp(logits - m), axis=-1, keepdims=True)) + m
    yv = y3_ref[...].reshape(_R, 1)
    onehot = lax.broadcasted_iota(jnp.int32, (_R, CPAD), 1) == yv
    ly = jnp.sum(jnp.where(onehot, logits, 0.0), axis=-1, keepdims=True)
    ae_part = jnp.sum(lse - ly)
    part = (ae_part / N + ass_part / (N * G)).reshape(1, 1)

    @pl.when(i == 0)
    def _():
        loss_ref[...] = jnp.zeros((1, 1), jnp.float32)

    loss_ref[...] += part


def _tc_stage(tok, pe, s1h, yemb, y3, b_W, b_b2, h_Wp, h_bp, segTp):
    nL = L // _R
    return pl.pallas_call(
        _tc_body,
        grid=(_NBLK,),
        in_specs=[
            pl.BlockSpec((_R, D), lambda i: (i, 0)),
            pl.BlockSpec((_R, D), lambda i: (i % nL, 0)),
            pl.BlockSpec((_R, 8), lambda i: (i, 0)),
            pl.BlockSpec((_R, G), lambda i: (i, 0)),
            pl.BlockSpec((1, 1, _R), lambda i: (i, 0, 0)),
            pl.BlockSpec((D, G), lambda i: (0, 0)),
            pl.BlockSpec((1, G), lambda i: (0, 0)),
            pl.BlockSpec((G, CPAD), lambda i: (0, 0)),
            pl.BlockSpec((1, CPAD), lambda i: (0, 0)),
            pl.BlockSpec((8, D), lambda i: (0, 0)),
        ],
        out_specs=[
            pl.BlockSpec((_R, D), lambda i: (i, 0)),
            pl.BlockSpec((1, 1), lambda i: (0, 0)),
        ],
        out_shape=[
            jax.ShapeDtypeStruct((N, D), jnp.float32),
            jax.ShapeDtypeStruct((1, 1), jnp.float32),
        ],
    )(tok, pe, s1h, yemb, y3, b_W, b_b2, h_Wp, h_bp, segTp)


def kernel(token_table, seg_table, g_table, b_W, b_b, h_W, h_b, sequence, segment_label, y):
    seq_flat = sequence.reshape(-1).astype(jnp.int32)
    y_flat = y.reshape(-1).astype(jnp.int32)
    seg_flat = segment_label.reshape(-1).astype(jnp.int32)

    tok, yemb = _sc_gather(token_table, g_table, seq_flat, y_flat)

    pe = jnp.asarray(_make_pe(L, D))
    s1h = (seg_flat[:, None] == jnp.arange(8, dtype=jnp.int32)[None, :]).astype(jnp.float32)
    segTp = jnp.zeros((8, D), jnp.float32).at[:3].set(seg_table)
    h_Wp = jnp.zeros((G, CPAD), jnp.float32).at[:, :CLASS].set(h_W)
    h_bp = jnp.full((1, CPAD), _NEG, jnp.float32).at[0, :CLASS].set(h_b)
    b_b2 = b_b.reshape(1, G)
    y3 = y_flat.reshape(_NBLK, 1, _R)

    x, loss = _tc_stage(tok, pe, s1h, yemb, y3, b_W, b_b2, h_Wp, h_bp, segTp)
    return (x.reshape(B, L, D), loss[0, 0])


# speedup vs baseline: 2.0655x; 1.0194x over previous
import functools

import jax
import jax.numpy as jnp
import numpy as np
from jax import lax
from jax.experimental import pallas as pl
from jax.experimental.pallas import tpu as pltpu
from jax.experimental.pallas import tpu_sc as plsc

VOCAB = 30522
D = 768
CLASS = 1000
CPAD = 1024
G = 128
B = 4
L = 2048
N = B * L

_NEG = -1e30


def _make_pe(seq_len, d_model):
    pos = np.arange(seq_len)[:, None].astype(np.float32)
    div = np.exp(np.arange(0, d_model, 2).astype(np.float32) * -(np.log(10000.0) / d_model))
    pe = np.zeros((seq_len, d_model), dtype=np.float32)
    pe[:, 0::2] = np.sin(pos * div)
    pe[:, 1::2] = np.cos(pos * div)
    return pe


def _sc_gather(token_table, g_table, seq_flat, y_flat):
    info = plsc.get_sparse_core_info()
    NC, NS = info.num_cores, info.num_subcores
    NW = NC * NS
    n_per_w = N // NW
    TOK_CH = 64
    Y_CH = 128

    mesh = plsc.VectorSubcoreMesh(core_axis_name="c", subcore_axis_name="s")

    @functools.partial(
        pl.kernel,
        mesh=mesh,
        out_type=(
            jax.ShapeDtypeStruct((N, D), jnp.float32),
            jax.ShapeDtypeStruct((N, G), jnp.float32),
        ),
        scratch_types=[
            pltpu.VMEM((n_per_w,), jnp.int32),
            pltpu.VMEM((n_per_w,), jnp.int32),
            pltpu.VMEM((TOK_CH, D), jnp.float32),
            pltpu.VMEM((Y_CH, G), jnp.float32),
            pltpu.SemaphoreType.DMA,
        ],
    )
    def k(tab_hbm, g_hbm, seq_hbm, y_hbm, tok_out, yemb_out,
          seq_v, y_v, tok_buf, y_buf, sem):
        wid = lax.axis_index("s") * NC + lax.axis_index("c")
        base = wid * n_per_w
        pltpu.sync_copy(seq_hbm.at[pl.ds(base, n_per_w)], seq_v)
        pltpu.sync_copy(y_hbm.at[pl.ds(base, n_per_w)], y_v)
        for c in range(n_per_w // TOK_CH):
            pltpu.async_copy(
                tab_hbm.at[seq_v.at[pl.ds(c * TOK_CH, TOK_CH)]], tok_buf, sem
            ).wait()
            pltpu.sync_copy(tok_buf, tok_out.at[pl.ds(base + c * TOK_CH, TOK_CH)])
        for c in range(n_per_w // Y_CH):
            pltpu.async_copy(
                g_hbm.at[y_v.at[pl.ds(c * Y_CH, Y_CH)]], y_buf, sem
            ).wait()
            pltpu.sync_copy(y_buf, yemb_out.at[pl.ds(base + c * Y_CH, Y_CH)])

    return k(token_table, g_table, seq_flat, y_flat)


_R = 256
_NBLK = N // _R


def _tc_body(tok_ref, pe_ref, s1h_ref, yemb_ref, y3_ref,
             bW_ref, bb_ref, hW_ref, hb_ref, segT_ref,
             x_ref, loss_ref):
    l, b = pl.program_id(0), pl.program_id(1)
    seg = jnp.dot(s1h_ref[...], segT_ref[...], preferred_element_type=jnp.float32)
    x = tok_ref[...] + pe_ref[...] + seg
    x_ref[...] = x
    yemb = yemb_ref[...]
    xb = jnp.dot(x.astype(jnp.bfloat16), bW_ref[...].astype(jnp.bfloat16),
                 preferred_element_type=jnp.float32) + bb_ref[...]
    diff = xb - yemb
    ass_part = jnp.sum(diff * diff)
    logits = jnp.dot(yemb.astype(jnp.bfloat16), hW_ref[...].astype(jnp.bfloat16),
                     preferred_element_type=jnp.float32) + hb_ref[...]
    m = jnp.max(logits, axis=-1, keepdims=True)
    lse = jnp.log(jnp.sum(jnp.exp(logits - m), axis=-1, keepdims=True)) + m
    yv = y3_ref[...].reshape(_R, 1)
    onehot = lax.broadcasted_iota(jnp.int32, (_R, CPAD), 1) == yv
    ly = jnp.sum(jnp.where(onehot, logits, 0.0), axis=-1, keepdims=True)
    ae_part = jnp.sum(lse - ly)
    part = (ae_part / N + ass_part / (N * G)).reshape(1, 1)

    @pl.when((l == 0) & (b == 0))
    def _():
        loss_ref[...] = jnp.zeros((1, 1), jnp.float32)

    loss_ref[...] += part


def _tc_stage(tok, pe, s1h, yemb, y3, b_W, b_b2, h_Wp, h_bp, segTp):
    nL = L // _R
    row = lambda l, b: b * nL + l
    return pl.pallas_call(
        _tc_body,
        grid=(nL, B),
        in_specs=[
            pl.BlockSpec((_R, D), lambda l, b: (row(l, b), 0)),
            pl.BlockSpec((_R, D), lambda l, b: (l, 0)),
            pl.BlockSpec((_R, 8), lambda l, b: (row(l, b), 0)),
            pl.BlockSpec((_R, G), lambda l, b: (row(l, b), 0)),
            pl.BlockSpec((1, 1, _R), lambda l, b: (row(l, b), 0, 0)),
            pl.BlockSpec((D, G), lambda l, b: (0, 0)),
            pl.BlockSpec((1, G), lambda l, b: (0, 0)),
            pl.BlockSpec((G, CPAD), lambda l, b: (0, 0)),
            pl.BlockSpec((1, CPAD), lambda l, b: (0, 0)),
            pl.BlockSpec((8, D), lambda l, b: (0, 0)),
        ],
        out_specs=[
            pl.BlockSpec((_R, D), lambda l, b: (row(l, b), 0)),
            pl.BlockSpec((1, 1), lambda l, b: (0, 0)),
        ],
        out_shape=[
            jax.ShapeDtypeStruct((N, D), jnp.float32),
            jax.ShapeDtypeStruct((1, 1), jnp.float32),
        ],
    )(tok, pe, s1h, yemb, y3, b_W, b_b2, h_Wp, h_bp, segTp)


def kernel(token_table, seg_table, g_table, b_W, b_b, h_W, h_b, sequence, segment_label, y):
    seq_flat = sequence.reshape(-1).astype(jnp.int32)
    y_flat = y.reshape(-1).astype(jnp.int32)
    seg_flat = segment_label.reshape(-1).astype(jnp.int32)

    tok, yemb = _sc_gather(token_table, g_table, seq_flat, y_flat)

    pe = jnp.asarray(_make_pe(L, D))
    s1h = (seg_flat[:, None] == jnp.arange(8, dtype=jnp.int32)[None, :]).astype(jnp.float32)
    segTp = jnp.zeros((8, D), jnp.float32).at[:3].set(seg_table)
    h_Wp = jnp.zeros((G, CPAD), jnp.float32).at[:, :CLASS].set(h_W)
    h_bp = jnp.full((1, CPAD), _NEG, jnp.float32).at[0, :CLASS].set(h_b)
    b_b2 = b_b.reshape(1, G)
    y3 = y_flat.reshape(_NBLK, 1, _R)

    x, loss = _tc_stage(tok, pe, s1h, yemb, y3, b_W, b_b2, h_Wp, h_bp, segTp)
    return (x.reshape(B, L, D), loss[0, 0])


# speedup vs baseline: 2.3763x; 1.1504x over previous
import functools

import jax
import jax.numpy as jnp
import numpy as np
from jax import lax
from jax.experimental import pallas as pl
from jax.experimental.pallas import tpu as pltpu
from jax.experimental.pallas import tpu_sc as plsc

VOCAB = 30522
D = 768
CLASS = 1000
CPAD = 1024
G = 128
B = 4
L = 2048
N = B * L

_NEG = -1e30


def _make_pe(seq_len, d_model):
    pos = np.arange(seq_len)[:, None].astype(np.float32)
    div = np.exp(np.arange(0, d_model, 2).astype(np.float32) * -(np.log(10000.0) / d_model))
    pe = np.zeros((seq_len, d_model), dtype=np.float32)
    pe[:, 0::2] = np.sin(pos * div)
    pe[:, 1::2] = np.cos(pos * div)
    return pe


def _sc_gather(token_table, g_table, seq_flat, y_flat):
    info = plsc.get_sparse_core_info()
    NC, NS = info.num_cores, info.num_subcores
    NW = NC * NS
    n_per_w = N // NW
    TOK_CH = 64
    Y_CH = 128

    mesh = plsc.VectorSubcoreMesh(core_axis_name="c", subcore_axis_name="s")

    @functools.partial(
        pl.kernel,
        mesh=mesh,
        out_type=(
            jax.ShapeDtypeStruct((N, D), jnp.float32),
            jax.ShapeDtypeStruct((N, G), jnp.float32),
        ),
        scratch_types=[
            pltpu.VMEM((n_per_w,), jnp.int32),
            pltpu.VMEM((n_per_w,), jnp.int32),
            pltpu.VMEM((TOK_CH, D), jnp.float32),
            pltpu.VMEM((Y_CH, G), jnp.float32),
            pltpu.SemaphoreType.DMA,
        ],
    )
    def k(tab_hbm, g_hbm, seq_hbm, y_hbm, tok_out, yemb_out,
          seq_v, y_v, tok_buf, y_buf, sem):
        wid = lax.axis_index("s") * NC + lax.axis_index("c")
        base = wid * n_per_w
        pltpu.sync_copy(seq_hbm.at[pl.ds(base, n_per_w)], seq_v)
        pltpu.sync_copy(y_hbm.at[pl.ds(base, n_per_w)], y_v)
        for c in range(n_per_w // TOK_CH):
            pltpu.async_copy(
                tab_hbm.at[seq_v.at[pl.ds(c * TOK_CH, TOK_CH)]], tok_buf, sem
            ).wait()
            pltpu.sync_copy(tok_buf, tok_out.at[pl.ds(base + c * TOK_CH, TOK_CH)])
        for c in range(n_per_w // Y_CH):
            pltpu.async_copy(
                g_hbm.at[y_v.at[pl.ds(c * Y_CH, Y_CH)]], y_buf, sem
            ).wait()
            pltpu.sync_copy(y_buf, yemb_out.at[pl.ds(base + c * Y_CH, Y_CH)])

    return k(token_table, g_table, seq_flat, y_flat)


_R = 512
_NBLK = N // _R


def _tc_body(tok_ref, pe_ref, s1h_ref, yemb_ref, y3_ref,
             bW_ref, bb_ref, hW_ref, hb_ref, segT_ref,
             x_ref, loss_ref):
    l, b = pl.program_id(0), pl.program_id(1)
    seg = jnp.dot(s1h_ref[...], segT_ref[...], preferred_element_type=jnp.float32)
    x = tok_ref[...] + pe_ref[...] + seg
    x_ref[...] = x
    yemb = yemb_ref[...]
    xb = jnp.dot(x.astype(jnp.bfloat16), bW_ref[...].astype(jnp.bfloat16),
                 preferred_element_type=jnp.float32) + bb_ref[...]
    diff = xb - yemb
    ass_part = jnp.sum(diff * diff)
    logits = jnp.dot(yemb.astype(jnp.bfloat16), hW_ref[...].astype(jnp.bfloat16),
                     preferred_element_type=jnp.float32) + hb_ref[...]
    m = jnp.max(logits, axis=-1, keepdims=True)
    lse = jnp.log(jnp.sum(jnp.exp(logits - m), axis=-1, keepdims=True)) + m
    yv = y3_ref[...].reshape(_R, 1)
    onehot = lax.broadcasted_iota(jnp.int32, (_R, CPAD), 1) == yv
    ly = jnp.sum(jnp.where(onehot, logits, 0.0), axis=-1, keepdims=True)
    ae_part = jnp.sum(lse - ly)
    part = (ae_part / N + ass_part / (N * G)).reshape(1, 1)

    @pl.when((l == 0) & (b == 0))
    def _():
        loss_ref[...] = jnp.zeros((1, 1), jnp.float32)

    loss_ref[...] += part


def _tc_stage(tok, pe, s1h, yemb, y3, b_W, b_b2, h_Wp, h_bp, segTp):
    nL = L // _R
    row = lambda l, b: b * nL + l
    return pl.pallas_call(
        _tc_body,
        grid=(nL, B),
        in_specs=[
            pl.BlockSpec((_R, D), lambda l, b: (row(l, b), 0)),
            pl.BlockSpec((_R, D), lambda l, b: (l, 0)),
            pl.BlockSpec((_R, 8), lambda l, b: (row(l, b), 0)),
            pl.BlockSpec((_R, G), lambda l, b: (row(l, b), 0)),
            pl.BlockSpec((1, 1, _R), lambda l, b: (row(l, b), 0, 0)),
            pl.BlockSpec((D, G), lambda l, b: (0, 0)),
            pl.BlockSpec((1, G), lambda l, b: (0, 0)),
            pl.BlockSpec((G, CPAD), lambda l, b: (0, 0)),
            pl.BlockSpec((1, CPAD), lambda l, b: (0, 0)),
            pl.BlockSpec((8, D), lambda l, b: (0, 0)),
        ],
        out_specs=[
            pl.BlockSpec((_R, D), lambda l, b: (row(l, b), 0)),
            pl.BlockSpec((1, 1), lambda l, b: (0, 0)),
        ],
        out_shape=[
            jax.ShapeDtypeStruct((N, D), jnp.float32),
            jax.ShapeDtypeStruct((1, 1), jnp.float32),
        ],
    )(tok, pe, s1h, yemb, y3, b_W, b_b2, h_Wp, h_bp, segTp)


def kernel(token_table, seg_table, g_table, b_W, b_b, h_W, h_b, sequence, segment_label, y):
    seq_flat = sequence.reshape(-1).astype(jnp.int32)
    y_flat = y.reshape(-1).astype(jnp.int32)
    seg_flat = segment_label.reshape(-1).astype(jnp.int32)

    tok, yemb = _sc_gather(token_table, g_table, seq_flat, y_flat)

    pe = jnp.asarray(_make_pe(L, D))
    s1h = (seg_flat[:, None] == jnp.arange(8, dtype=jnp.int32)[None, :]).astype(jnp.float32)
    segTp = jnp.zeros((8, D), jnp.float32).at[:3].set(seg_table)
    h_Wp = jnp.zeros((G, CPAD), jnp.float32).at[:, :CLASS].set(h_W)
    h_bp = jnp.full((1, CPAD), _NEG, jnp.float32).at[0, :CLASS].set(h_b)
    b_b2 = b_b.reshape(1, G)
    y3 = y_flat.reshape(_NBLK, 1, _R)

    x, loss = _tc_stage(tok, pe, s1h, yemb, y3, b_W, b_b2, h_Wp, h_bp, segTp)
    return (x.reshape(B, L, D), loss[0, 0])


# speedup vs baseline: 2.4970x; 1.0508x over previous
import functools

import jax
import jax.numpy as jnp
import numpy as np
from jax import lax
from jax.experimental import pallas as pl
from jax.experimental.pallas import tpu as pltpu
from jax.experimental.pallas import tpu_sc as plsc

VOCAB = 30522
D = 768
CLASS = 1000
CPAD = 1024
G = 128
B = 4
L = 2048
N = B * L

_NEG = -1e30


def _make_pe(seq_len, d_model):
    pos = np.arange(seq_len)[:, None].astype(np.float32)
    div = np.exp(np.arange(0, d_model, 2).astype(np.float32) * -(np.log(10000.0) / d_model))
    pe = np.zeros((seq_len, d_model), dtype=np.float32)
    pe[:, 0::2] = np.sin(pos * div)
    pe[:, 1::2] = np.cos(pos * div)
    return pe


def _sc_gather(token_table, g_table, seq_flat, y_flat):
    info = plsc.get_sparse_core_info()
    NC, NS = info.num_cores, info.num_subcores
    NW = NC * NS
    n_per_w = N // NW
    TOK_CH = 64
    Y_CH = 128

    mesh = plsc.VectorSubcoreMesh(core_axis_name="c", subcore_axis_name="s")

    @functools.partial(
        pl.kernel,
        mesh=mesh,
        out_type=(
            jax.ShapeDtypeStruct((N, D), jnp.float32),
            jax.ShapeDtypeStruct((N, G), jnp.float32),
        ),
        scratch_types=[
            pltpu.VMEM((n_per_w,), jnp.int32),
            pltpu.VMEM((n_per_w,), jnp.int32),
            pltpu.VMEM((TOK_CH, D), jnp.float32),
            pltpu.VMEM((Y_CH, G), jnp.float32),
            pltpu.SemaphoreType.DMA,
        ],
    )
    def k(tab_hbm, g_hbm, seq_hbm, y_hbm, tok_out, yemb_out,
          seq_v, y_v, tok_buf, y_buf, sem):
        wid = lax.axis_index("s") * NC + lax.axis_index("c")
        base = wid * n_per_w
        pltpu.sync_copy(seq_hbm.at[pl.ds(base, n_per_w)], seq_v)
        pltpu.sync_copy(y_hbm.at[pl.ds(base, n_per_w)], y_v)
        for c in range(n_per_w // TOK_CH):
            pltpu.async_copy(
                tab_hbm.at[seq_v.at[pl.ds(c * TOK_CH, TOK_CH)]], tok_buf, sem
            ).wait()
            pltpu.sync_copy(tok_buf, tok_out.at[pl.ds(base + c * TOK_CH, TOK_CH)])
        for c in range(n_per_w // Y_CH):
            pltpu.async_copy(
                g_hbm.at[y_v.at[pl.ds(c * Y_CH, Y_CH)]], y_buf, sem
            ).wait()
            pltpu.sync_copy(y_buf, yemb_out.at[pl.ds(base + c * Y_CH, Y_CH)])

    return k(token_table, g_table, seq_flat, y_flat)


_R = 1024
_NBLK = N // _R


def _tc_body(tok_ref, pe_ref, s1h_ref, yemb_ref, y3_ref,
             bW_ref, bb_ref, hW_ref, hb_ref, segT_ref,
             x_ref, loss_ref):
    l, b = pl.program_id(0), pl.program_id(1)
    seg = jnp.dot(s1h_ref[...], segT_ref[...], preferred_element_type=jnp.float32)
    x = tok_ref[...] + pe_ref[...] + seg
    x_ref[...] = x
    yemb = yemb_ref[...]
    xb = jnp.dot(x.astype(jnp.bfloat16), bW_ref[...].astype(jnp.bfloat16),
                 preferred_element_type=jnp.float32) + bb_ref[...]
    diff = xb - yemb
    ass_part = jnp.sum(diff * diff)
    logits = jnp.dot(yemb.astype(jnp.bfloat16), hW_ref[...].astype(jnp.bfloat16),
                     preferred_element_type=jnp.float32) + hb_ref[...]
    m = jnp.max(logits, axis=-1, keepdims=True)
    lse = jnp.log(jnp.sum(jnp.exp(logits - m), axis=-1, keepdims=True)) + m
    yv = y3_ref[...].reshape(_R, 1)
    onehot = lax.broadcasted_iota(jnp.int32, (_R, CPAD), 1) == yv
    ly = jnp.sum(jnp.where(onehot, logits, 0.0), axis=-1, keepdims=True)
    ae_part = jnp.sum(lse - ly)
    part = (ae_part / N + ass_part / (N * G)).reshape(1, 1)

    @pl.when((l == 0) & (b == 0))
    def _():
        loss_ref[...] = jnp.zeros((1, 1), jnp.float32)

    loss_ref[...] += part


def _tc_stage(tok, pe, s1h, yemb, y3, b_W, b_b2, h_Wp, h_bp, segTp):
    nL = L // _R
    row = lambda l, b: b * nL + l
    return pl.pallas_call(
        _tc_body,
        grid=(nL, B),
        in_specs=[
            pl.BlockSpec((_R, D), lambda l, b: (row(l, b), 0)),
            pl.BlockSpec((_R, D), lambda l, b: (l, 0)),
            pl.BlockSpec((_R, 8), lambda l, b: (row(l, b), 0)),
            pl.BlockSpec((_R, G), lambda l, b: (row(l, b), 0)),
            pl.BlockSpec((1, 1, _R), lambda l, b: (row(l, b), 0, 0)),
            pl.BlockSpec((D, G), lambda l, b: (0, 0)),
            pl.BlockSpec((1, G), lambda l, b: (0, 0)),
            pl.BlockSpec((G, CPAD), lambda l, b: (0, 0)),
            pl.BlockSpec((1, CPAD), lambda l, b: (0, 0)),
            pl.BlockSpec((8, D), lambda l, b: (0, 0)),
        ],
        out_specs=[
            pl.BlockSpec((_R, D), lambda l, b: (row(l, b), 0)),
            pl.BlockSpec((1, 1), lambda l, b: (0, 0)),
        ],
        out_shape=[
            jax.ShapeDtypeStruct((N, D), jnp.float32),
            jax.ShapeDtypeStruct((1, 1), jnp.float32),
        ],
    )(tok, pe, s1h, yemb, y3, b_W, b_b2, h_Wp, h_bp, segTp)


def kernel(token_table, seg_table, g_table, b_W, b_b, h_W, h_b, sequence, segment_label, y):
    seq_flat = sequence.reshape(-1).astype(jnp.int32)
    y_flat = y.reshape(-1).astype(jnp.int32)
    seg_flat = segment_label.reshape(-1).astype(jnp.int32)

    tok, yemb = _sc_gather(token_table, g_table, seq_flat, y_flat)

    pe = jnp.asarray(_make_pe(L, D))
    s1h = (seg_flat[:, None] == jnp.arange(8, dtype=jnp.int32)[None, :]).astype(jnp.float32)
    segTp = jnp.zeros((8, D), jnp.float32).at[:3].set(seg_table)
    h_Wp = jnp.zeros((G, CPAD), jnp.float32).at[:, :CLASS].set(h_W)
    h_bp = jnp.full((1, CPAD), _NEG, jnp.float32).at[0, :CLASS].set(h_b)
    b_b2 = b_b.reshape(1, G)
    y3 = y_flat.reshape(_NBLK, 1, _R)

    x, loss = _tc_stage(tok, pe, s1h, yemb, y3, b_W, b_b2, h_Wp, h_bp, segTp)
    return (x.reshape(B, L, D), loss[0, 0])
